# Initial kernel scaffold; baseline (speedup 1.0000x reference)
import jax
import jax.numpy as jnp
from jax.experimental import pallas as pl


def kernel(positions_x, positions_y, pe):
    raise NotImplementedError("write your pallas kernel here")


# speedup vs baseline: 11.3276x; 11.3276x over previous
import functools

import jax
import jax.numpy as jnp
from jax import lax
from jax.experimental import pallas as pl
from jax.experimental.pallas import tpu as pltpu
from jax.experimental.pallas import tpu_sc as plsc

D_MODEL = 128
N_ROWS = 256

NC = 2
NS = 16
L = 16
NW = NC * NS

_B = 4096 * 50
_PER_W = _B // NW
_CH = 128
_NCHUNK = _PER_W // _CH


def _sc_gather(px, py, pe_flat):
    mesh = plsc.VectorSubcoreMesh(core_axis_name="c", subcore_axis_name="s")

    @functools.partial(
        pl.kernel,
        mesh=mesh,
        out_type=jax.ShapeDtypeStruct((_B, D_MODEL), jnp.float32),
        scratch_types=[
            pltpu.VMEM((_PER_W,), jnp.int32),
            pltpu.VMEM((_PER_W,), jnp.int32),
            pltpu.VMEM((_CH,), jnp.int32),
            pltpu.VMEM((_CH, D_MODEL), jnp.float32),
            pltpu.SemaphoreType.DMA,
        ],
    )
    def k(px_hbm, py_hbm, pe_hbm, out_hbm, pxv, pyv, idxc, rows, sem):
        wid = lax.axis_index("s") * NC + lax.axis_index("c")
        base = wid * _PER_W
        pltpu.sync_copy(px_hbm.at[pl.ds(base, _PER_W)], pxv)
        pltpu.sync_copy(py_hbm.at[pl.ds(base, _PER_W)], pyv)

        def chunk(c, carry):
            co = c * _CH

            def vec(j, carry2):
                o = co + j * L
                x = pxv[pl.ds(o, L)]
                y = pyv[pl.ds(o, L)]
                idxc[pl.ds(j * L, L)] = x * N_ROWS + y
                return carry2

            lax.fori_loop(0, _CH // L, vec, 0, unroll=True)
            pltpu.async_copy(pe_hbm.at[idxc], rows, sem).wait()
            pltpu.sync_copy(rows, out_hbm.at[pl.ds(base + co, _CH)])
            return carry

        lax.fori_loop(0, _NCHUNK, chunk, 0)

    return k(px, py, pe_flat)


def kernel(positions_x, positions_y, pe):
    B, S = positions_x.shape
    px = positions_x.reshape(-1).astype(jnp.int32)
    py = positions_y.reshape(-1).astype(jnp.int32)
    pe_flat = pe.reshape(N_ROWS * N_ROWS, D_MODEL)
    out = _sc_gather(px, py, pe_flat)
    return out.reshape(B, S, D_MODEL)
